# baseline (speedup 1.0000x reference)
import functools

import jax
import jax.numpy as jnp
from jax import lax
from jax.experimental import pallas as pl
from jax.experimental.pallas import tpu as pltpu
from jax.experimental.pallas import tpu_sc as plsc

D = 100
LANES = 16
COL_OFFS = (0, 16, 32, 48, 64, 80, 84)
NACC = len(COL_OFFS)
ACC_W = NACC * LANES

NC = 2
NS = 16
NW = NC * NS

V = 1000000
NBLK8 = V // 8
CH = 256
NCHUNK = 124
IDX_CH = 8192
SLICE_MAX = 32064
SENTINEL = NCHUNK * CH + 16


def _slice_lo(t):
    return (t * NBLK8 // NW) * 8


def _sc_embed_sum(input_idx, weight):
    B = input_idx.shape[0]

    mesh = plsc.VectorSubcoreMesh(core_axis_name="c", subcore_axis_name="s")

    @functools.partial(
        pl.kernel,
        out_type=jax.ShapeDtypeStruct((NW, ACC_W), jnp.float32),
        mesh=mesh,
        compiler_params=pltpu.CompilerParams(needs_layout_passes=False),
        scratch_types=[
            pltpu.VMEM((IDX_CH,), jnp.int32),
            pltpu.VMEM((B + LANES,), jnp.int32),
            pltpu.VMEM((SLICE_MAX,), jnp.float32),
            pltpu.VMEM((2, CH, D), jnp.float32),
            pltpu.VMEM((ACC_W,), jnp.float32),
            pltpu.SemaphoreType.DMA,
            pltpu.SemaphoreType.DMA,
        ],
    )
    def k(idx_hbm, tbl_hbm, out_hbm, idx_v, list_v, cnt_v, bufs_v, acc_v,
          sem0, sem1):
        cid = lax.axis_index("c")
        sid = lax.axis_index("s")
        wid = sid * NC + cid
        lo = _slice_lo(wid)
        size = _slice_lo(wid + 1) - lo
        buf_lo = jnp.minimum(lo, V - NCHUNK * CH)
        mlo = lo - buf_lo

        def scan_body(g, ptr):
            vv = idx_v[pl.ds(g * LANES, LANES)]
            r = vv - buf_lo
            m = (r >= mlo) & (r < mlo + size)
            pos = ptr + plsc.cumsum(m.astype(jnp.int32)) - 1
            plsc.store_scatter(list_v, [pos], r, mask=m)
            return ptr + plsc.all_reduce_population_count(m)[0]

        nloc = 0
        for h in range(B // IDX_CH):
            pltpu.sync_copy(idx_hbm.at[pl.ds(h * IDX_CH, IDX_CH)], idx_v)
            nloc = lax.fori_loop(0, IDX_CH // LANES, scan_body, nloc)
        list_v[pl.ds(nloc, LANES)] = jnp.full((LANES,), SENTINEL, jnp.int32)

        def zero_body(i, _):
            cnt_v[pl.ds(i * LANES, LANES)] = jnp.zeros((LANES,), jnp.float32)
            return 0
        lax.fori_loop(0, SLICE_MAX // LANES, zero_body, 0)

        one0 = (lax.iota(jnp.int32, LANES) == 0).astype(jnp.float32)

        def cnt_body(q, _):
            w = list_v[pl.ds(q * LANES, LANES)]
            for j in range(LANES):
                r = w[j]
                cnt_v[pl.ds(r, LANES)] = cnt_v[pl.ds(r, LANES)] + one0
            return 0
        lax.fori_loop(0, (nloc + LANES - 1) // LANES, cnt_body, 0)

        def chunk_src(c):
            return tbl_hbm.at[pl.ds(buf_lo + c * CH, CH)]

        for i in range(NACC):
            acc_v[pl.ds(i * LANES, LANES)] = jnp.zeros((LANES,), jnp.float32)

        sems = (sem0, sem1)
        pltpu.async_copy(chunk_src(0), bufs_v.at[0], sem0)
        pltpu.async_copy(chunk_src(1), bufs_v.at[1], sem1)

        def acc_chunk(c, p):
            pltpu.make_async_copy(chunk_src(c), bufs_v.at[p], sems[p]).wait()
            for g in range(CH // LANES):
                cv = cnt_v[pl.ds(c * CH + g * LANES, LANES)]

                @pl.when(jnp.max(cv) > 0.0)
                def _():
                    def row_body(j, a):
                        cw = jnp.take(cv, jnp.full((LANES,), j, jnp.int32))
                        row = g * LANES + j
                        return tuple(
                            a[i] + cw * bufs_v[p, row,
                                               pl.ds(COL_OFFS[i], LANES)]
                            for i in range(NACC)
                        )

                    a0 = tuple(acc_v[pl.ds(i * LANES, LANES)]
                               for i in range(NACC))
                    a = lax.fori_loop(0, LANES, row_body, a0)
                    for i in range(NACC):
                        acc_v[pl.ds(i * LANES, LANES)] = a[i]

        def pipe_body(h, carry):
            c0 = h * 2
            acc_chunk(c0, 0)

            @pl.when(c0 + 2 < NCHUNK)
            def _():
                pltpu.async_copy(chunk_src(c0 + 2), bufs_v.at[0], sem0)

            acc_chunk(c0 + 1, 1)

            @pl.when(c0 + 3 < NCHUNK)
            def _():
                pltpu.async_copy(chunk_src(c0 + 3), bufs_v.at[1], sem1)

            return carry

        lax.fori_loop(0, NCHUNK // 2, pipe_body, 0)

        pltpu.sync_copy(acc_v, out_hbm.at[wid])

    return k(input_idx, weight)


def kernel(input, weight):
    part = _sc_embed_sum(input.astype(jnp.int32), weight)
    w = part.sum(axis=0)
    return jnp.concatenate([w[:96], w[108:112]])


# speedup vs baseline: 1.4980x; 1.4980x over previous
import functools

import jax
import jax.numpy as jnp
from jax import lax
from jax.experimental import pallas as pl
from jax.experimental.pallas import tpu as pltpu
from jax.experimental.pallas import tpu_sc as plsc

D = 100
LANES = 16
COL_OFFS = (0, 16, 32, 48, 64, 80, 84)
NACC = len(COL_OFFS)
ACC_W = NACC * LANES

NC = 2
NS = 16
NW = NC * NS

GRP = 64


def _sc_embed_sum(input_idx, weight):
    B = input_idx.shape[0]
    BPW = B // NW
    NGRP = BPW // GRP

    mesh = plsc.VectorSubcoreMesh(core_axis_name="c", subcore_axis_name="s")

    @functools.partial(
        pl.kernel,
        out_type=jax.ShapeDtypeStruct((NW, ACC_W), jnp.float32),
        mesh=mesh,
        scratch_types=[
            pltpu.VMEM((BPW,), jnp.int32),
            pltpu.VMEM((GRP, D), jnp.float32),
            pltpu.VMEM((ACC_W,), jnp.float32),
            pltpu.SemaphoreType.DMA,
            pltpu.SemaphoreType.DMA,
            pltpu.SemaphoreType.DMA,
            pltpu.SemaphoreType.DMA,
        ],
    )
    def k(idx_hbm, tbl_hbm, out_hbm, idx_v, rows_v, acc_v, *sems):
        cid = lax.axis_index("c")
        sid = lax.axis_index("s")
        wid = sid * NC + cid
        base = wid * BPW

        pltpu.sync_copy(idx_hbm.at[pl.ds(base, BPW)], idx_v)

        def body(g, accs):
            v = idx_v[pl.ds(g * GRP, GRP)]
            for lane in range(GRP):
                pltpu.async_copy(tbl_hbm.at[v[lane]], rows_v.at[lane],
                                 sems[lane % 4])
            for q in range(4):
                pltpu.make_async_copy(
                    tbl_hbm.at[pl.ds(0, GRP // 4)],
                    rows_v.at[pl.ds(q * (GRP // 4), GRP // 4)],
                    sems[q]).wait()
            for lane in range(GRP):
                accs = tuple(
                    accs[i] + rows_v[lane, pl.ds(COL_OFFS[i], LANES)]
                    for i in range(NACC)
                )
            return accs

        zero = jnp.zeros((LANES,), jnp.float32)
        accs = lax.fori_loop(0, NGRP, body, (zero,) * NACC)

        for i in range(NACC):
            acc_v[pl.ds(i * LANES, LANES)] = accs[i]
        pltpu.sync_copy(acc_v, out_hbm.at[wid])

    return k(input_idx, weight)


def kernel(input, weight):
    part = _sc_embed_sum(input.astype(jnp.int32), weight)
    w = part.sum(axis=0)
    return jnp.concatenate([w[:96], w[108:112]])
